# baseline (speedup 1.0000x reference)
import functools

import jax
import jax.numpy as jnp
from jax import lax
from jax.experimental import pallas as pl
from jax.experimental.pallas import tpu as pltpu
from jax.experimental.pallas import tpu_sc as plsc

def _dyn_gather(v, idx):
  return lax.gather(
      v, idx[:, None],
      lax.GatherDimensionNumbers(
          offset_dims=(), collapsed_slice_dims=(0,), start_index_map=(0,)),
      (1,), mode=lax.GatherScatterMode.PROMISE_IN_BOUNDS)


B1 = 32
B2 = 8
N = 8192
LANES = 16
STEPS = N // LANES


def _argmin_kernel(x_hbm, out_hbm, buf, res_v, sem0, sem1):
  cid = lax.axis_index("c")
  sid = lax.axis_index("s")
  wid = sid * 2 + cid
  sems = [sem0, sem1]
  iota = lax.iota(jnp.int32, 16)

  pltpu.make_async_copy(x_hbm.at[wid, 0], buf.at[0], sem0).start()

  res = jnp.zeros((LANES,), jnp.int32)
  for r in range(B2):
    b = r % 2
    if r + 1 < B2:
      nb = (r + 1) % 2
      pltpu.make_async_copy(x_hbm.at[wid, r + 1], buf.at[nb], sems[nb]).start()
    pltpu.make_async_copy(x_hbm.at[wid, r], buf.at[b], sems[b]).wait()

    def body(i, carry):
      minv, mini, idxv = carry
      v = buf[b, pl.ds(i * LANES, LANES)]
      m = v < minv
      return (jnp.where(m, v, minv), jnp.where(m, idxv, mini), idxv + LANES)

    init = (jnp.full((LANES,), jnp.inf, jnp.float32),
            jnp.zeros((LANES,), jnp.int32), iota)
    minv, mini, _ = lax.fori_loop(0, STEPS, body, init, unroll=4)

    for d in (8, 4, 2, 1):
      perm = (iota + d) & (LANES - 1)
      v2 = _dyn_gather(minv, perm)
      i2 = _dyn_gather(mini, perm)
      take = (v2 < minv) | ((v2 == minv) & (i2 < mini))
      minv = jnp.where(take, v2, minv)
      mini = jnp.where(take, i2, mini)
    res = jnp.where(iota == r, mini, res)

  res_v[...] = res
  pltpu.sync_copy(res_v.at[pl.ds(0, B2)], out_hbm.at[pl.ds(wid * B2, B2)])


@jax.jit
def kernel(x):
  mesh = plsc.VectorSubcoreMesh(core_axis_name="c", subcore_axis_name="s")
  run = functools.partial(
      pl.kernel,
      mesh=mesh,
      out_type=jax.ShapeDtypeStruct((B1 * B2,), jnp.int32),
      scratch_types=[
          pltpu.VMEM((2, N), jnp.float32),
          pltpu.VMEM((LANES,), jnp.int32),
          pltpu.SemaphoreType.DMA,
          pltpu.SemaphoreType.DMA,
      ],
  )(_argmin_kernel)
  return run(x).reshape(B1, B2)


# speedup vs baseline: 1.1541x; 1.1541x over previous
import functools

import jax
import jax.numpy as jnp
from jax import lax
from jax.experimental import pallas as pl
from jax.experimental.pallas import tpu as pltpu
from jax.experimental.pallas import tpu_sc as plsc

B1 = 32
B2 = 8
N = 8192
LANES = 16
CHAINS = 4
GROUP = CHAINS * LANES
STEPS = N // GROUP


def _dyn_gather(v, idx):
  return lax.gather(
      v, idx[:, None],
      lax.GatherDimensionNumbers(
          offset_dims=(), collapsed_slice_dims=(0,), start_index_map=(0,)),
      (1,), mode=lax.GatherScatterMode.PROMISE_IN_BOUNDS)


def _lex_merge(v, i, v2, i2):
  take = (v2 < v) | ((v2 == v) & (i2 < i))
  return jnp.where(take, v2, v), jnp.where(take, i2, i)


def _argmin_kernel(x_hbm, out_hbm, buf, res_v, sem):
  cid = lax.axis_index("c")
  sid = lax.axis_index("s")
  wid = sid * 2 + cid
  iota = lax.iota(jnp.int32, LANES)

  for r in range(B2):
    pltpu.make_async_copy(x_hbm.at[wid, r], buf.at[r], sem).start()

  def row_body(r, res):
    pltpu.make_async_copy(x_hbm.at[wid, r], buf.at[r], sem).wait()

    def body(i, carry):
      minvs, minis = carry
      step = jnp.full((LANES,), i, jnp.int32)
      new_v, new_i = [], []
      for j in range(CHAINS):
        v = buf[r, pl.ds(i * GROUP + j * LANES, LANES)]
        m = v < minvs[j]
        new_v.append(jnp.minimum(v, minvs[j]))
        new_i.append(jnp.where(m, step, minis[j]))
      return tuple(new_v), tuple(new_i)

    init = (tuple(jnp.full((LANES,), jnp.inf, jnp.float32)
                  for _ in range(CHAINS)),
            tuple(jnp.zeros((LANES,), jnp.int32) for _ in range(CHAINS)))
    minvs, minis = lax.fori_loop(0, STEPS, body, init, unroll=4)

    mv, mi = None, None
    for j in range(CHAINS):
      ej = minis[j] * GROUP + (j * LANES) + iota
      if mv is None:
        mv, mi = minvs[j], ej
      else:
        mv, mi = _lex_merge(mv, mi, minvs[j], ej)

    for d in (8, 4, 2, 1):
      perm = (iota + d) & (LANES - 1)
      mv2 = _dyn_gather(mv, perm)
      mi2 = _dyn_gather(mi, perm)
      mv, mi = _lex_merge(mv, mi, mv2, mi2)

    return jnp.where(iota == r, mi, res)

  res = lax.fori_loop(0, B2, row_body, jnp.zeros((LANES,), jnp.int32))
  res_v[...] = res
  pltpu.sync_copy(res_v.at[pl.ds(0, B2)], out_hbm.at[pl.ds(wid * B2, B2)])


@jax.jit
def kernel(x):
  mesh = plsc.VectorSubcoreMesh(core_axis_name="c", subcore_axis_name="s")
  run = functools.partial(
      pl.kernel,
      mesh=mesh,
      out_type=jax.ShapeDtypeStruct((B1 * B2,), jnp.int32),
      scratch_types=[
          pltpu.VMEM((B2, N), jnp.float32),
          pltpu.VMEM((LANES,), jnp.int32),
          pltpu.SemaphoreType.DMA,
      ],
  )(_argmin_kernel)
  return run(x).reshape(B1, B2)
